# baseline (device time: 25339 ns/iter reference)
import jax
import jax.numpy as jnp
from jax import lax
from jax.experimental import pallas as pl
from jax.experimental.pallas import tpu as pltpu

N_DEV = 8


def kernel(x, w_mat):
    m_per, k = x.shape
    n = w_mat.shape[1]
    n_per = n // N_DEV

    def body(x_hbm, w_hbm, out_ref, w_vmem, x_f32, x_bf, send_buf, recv_buf,
             x_sem, w_sems, send_sems, recv_sems):
        me = lax.axis_index("i")

        x_dma = pltpu.make_async_copy(x_hbm, x_f32, x_sem)
        x_dma.start()

        def w_dma(d):
            slot = d - 1
            dst = (me + d) % N_DEV
            return pltpu.make_async_copy(
                w_hbm.at[:, pl.ds(dst * n_per, n_per)],
                w_vmem.at[slot],
                w_sems.at[slot],
            )

        for d in range(1, N_DEV + 1):
            w_dma(d).start()

        x_dma.wait()
        x_bf[:, :] = x_f32[:, :].astype(jnp.bfloat16)

        barrier_sem = pltpu.get_barrier_semaphore()
        for d in range(1, N_DEV):
            pl.semaphore_signal(
                barrier_sem, inc=1,
                device_id=((me + d) % N_DEV,),
                device_id_type=pl.DeviceIdType.MESH,
            )
        pl.semaphore_wait(barrier_sem, N_DEV - 1)

        for d in range(1, N_DEV + 1):
            w_dma(d).wait()
            blk = jnp.dot(
                x_bf[:, :],
                w_vmem[d - 1].astype(jnp.bfloat16),
                preferred_element_type=jnp.float32,
            )
            if d < N_DEV:
                dst = (me + d) % N_DEV
                send_buf[dst, :, :] = blk.astype(jnp.bfloat16)
                rdma = pltpu.make_async_remote_copy(
                    src_ref=send_buf.at[dst],
                    dst_ref=recv_buf.at[me],
                    send_sem=send_sems.at[dst],
                    recv_sem=recv_sems.at[me],
                    device_id=(dst,),
                    device_id_type=pl.DeviceIdType.MESH,
                )
                rdma.start()
            else:
                out_ref[pl.ds(me * m_per, m_per), :] = blk

        for d in range(1, N_DEV):
            src = (me - d) % N_DEV
            recv = pltpu.make_async_remote_copy(
                src_ref=recv_buf.at[src],
                dst_ref=recv_buf.at[src],
                send_sem=send_sems.at[src],
                recv_sem=recv_sems.at[src],
                device_id=(src,),
                device_id_type=pl.DeviceIdType.MESH,
            )
            recv.wait_recv()
            out_ref[pl.ds(src * m_per, m_per), :] = recv_buf[src].astype(
                jnp.float32
            )

        for d in range(1, N_DEV):
            dst = (me + d) % N_DEV
            done = pltpu.make_async_remote_copy(
                src_ref=send_buf.at[dst],
                dst_ref=recv_buf.at[me],
                send_sem=send_sems.at[dst],
                recv_sem=recv_sems.at[me],
                device_id=(dst,),
                device_id_type=pl.DeviceIdType.MESH,
            )
            done.wait_send()

    return pl.pallas_call(
        body,
        out_shape=jax.ShapeDtypeStruct((N_DEV * m_per, n_per), jnp.float32),
        in_specs=[
            pl.BlockSpec(memory_space=pltpu.MemorySpace.HBM),
            pl.BlockSpec(memory_space=pltpu.MemorySpace.HBM),
        ],
        out_specs=pl.BlockSpec(memory_space=pltpu.VMEM),
        scratch_shapes=[
            pltpu.VMEM((N_DEV, k, n_per), w_mat.dtype),
            pltpu.VMEM((m_per, k), x.dtype),
            pltpu.VMEM((m_per, k), jnp.bfloat16),
            pltpu.VMEM((N_DEV, m_per, n_per), jnp.bfloat16),
            pltpu.VMEM((N_DEV, m_per, n_per), jnp.bfloat16),
            pltpu.SemaphoreType.DMA,
            pltpu.SemaphoreType.DMA((N_DEV,)),
            pltpu.SemaphoreType.DMA((N_DEV,)),
            pltpu.SemaphoreType.DMA((N_DEV,)),
        ],
        compiler_params=pltpu.CompilerParams(collective_id=0),
    )(x, w_mat)


# device time: 22443 ns/iter; 1.1290x vs baseline; 1.1290x over previous
import jax
import jax.numpy as jnp
from jax import lax
from jax.experimental import pallas as pl
from jax.experimental.pallas import tpu as pltpu

N_DEV = 8


def kernel(x, w_mat):
    m_per, k = x.shape
    n = w_mat.shape[1]
    n_per = n // N_DEV

    def body(x_ref, w_ref, out_ref, x_bf, send_buf, recv_buf,
             send_sems, recv_sems):
        me = lax.axis_index("i")

        x_bf[:, :] = x_ref[:, :].astype(jnp.bfloat16)

        barrier_sem = pltpu.get_barrier_semaphore()
        for d in range(1, N_DEV):
            pl.semaphore_signal(
                barrier_sem, inc=1,
                device_id=((me + d) % N_DEV,),
                device_id_type=pl.DeviceIdType.MESH,
            )
        pl.semaphore_wait(barrier_sem, N_DEV - 1)

        for d in range(1, N_DEV + 1):
            dst = (me + d) % N_DEV
            blk = jnp.dot(
                x_bf[:, :],
                w_ref[:, pl.ds(dst * n_per, n_per)].astype(jnp.bfloat16),
                preferred_element_type=jnp.float32,
            )
            if d < N_DEV:
                send_buf[dst, :, :] = blk.astype(jnp.bfloat16)
                rdma = pltpu.make_async_remote_copy(
                    src_ref=send_buf.at[dst],
                    dst_ref=recv_buf.at[me],
                    send_sem=send_sems.at[dst],
                    recv_sem=recv_sems.at[me],
                    device_id=(dst,),
                    device_id_type=pl.DeviceIdType.MESH,
                )
                rdma.start()
            else:
                out_ref[pl.ds(me * m_per, m_per), :] = blk

        for d in range(1, N_DEV):
            src = (me - d) % N_DEV
            recv = pltpu.make_async_remote_copy(
                src_ref=recv_buf.at[src],
                dst_ref=recv_buf.at[src],
                send_sem=send_sems.at[src],
                recv_sem=recv_sems.at[src],
                device_id=(src,),
                device_id_type=pl.DeviceIdType.MESH,
            )
            recv.wait_recv()
            out_ref[pl.ds(src * m_per, m_per), :] = recv_buf[src].astype(
                jnp.float32
            )

        for d in range(1, N_DEV):
            dst = (me + d) % N_DEV
            done = pltpu.make_async_remote_copy(
                src_ref=send_buf.at[dst],
                dst_ref=recv_buf.at[me],
                send_sem=send_sems.at[dst],
                recv_sem=recv_sems.at[me],
                device_id=(dst,),
                device_id_type=pl.DeviceIdType.MESH,
            )
            done.wait_send()

    return pl.pallas_call(
        body,
        out_shape=jax.ShapeDtypeStruct((N_DEV * m_per, n_per), jnp.float32),
        in_specs=[
            pl.BlockSpec(memory_space=pltpu.VMEM),
            pl.BlockSpec(memory_space=pltpu.VMEM),
        ],
        out_specs=pl.BlockSpec(memory_space=pltpu.VMEM),
        scratch_shapes=[
            pltpu.VMEM((m_per, k), jnp.bfloat16),
            pltpu.VMEM((N_DEV, m_per, n_per), jnp.bfloat16),
            pltpu.VMEM((N_DEV, m_per, n_per), jnp.bfloat16),
            pltpu.SemaphoreType.DMA((N_DEV,)),
            pltpu.SemaphoreType.DMA((N_DEV,)),
        ],
        compiler_params=pltpu.CompilerParams(collective_id=0),
    )(x, w_mat)
